# Initial kernel scaffold; baseline (speedup 1.0000x reference)
import jax
import jax.numpy as jnp
from jax.experimental import pallas as pl


def kernel(s, v, p, e, batch, edge_index_global, W_shared, b_shared, W_bond, b_bond, W0, b0, W1, b1, Wc):
    raise NotImplementedError("write your pallas kernel here")


# speedup vs baseline: 12.1346x; 12.1346x over previous
import functools

import jax
import jax.numpy as jnp
from jax import lax
from jax.experimental import pallas as pl
from jax.experimental.pallas import tpu as pltpu
from jax.experimental.pallas import tpu_sc as plsc

_N = 2048
_E = 65536
_SDIM = 256
_VDIM = 64
_EDIM = 16
_NG = 64
_NBT = 5
_TEXT = 272
_NKEY = _N * _N
_NBUCKET = 64
_NW = 32
_EPW = _E // _NW
_GW = 128
_KW = 2048


def _silu(x):
    return x * (1.0 / (1.0 + jnp.exp(-x)))


def _node_body(s_ref, v2_ref, p_ref, b_ref, j2_ref, i2_ref, wsh_ref, bsh_ref,
               wc_ref, wb_ref, bb_ref, w0_ref, b0_ref,
               text_ref, k2_ref, keff2_ref, rk2_ref, m_ref, c0_ref, wd_ref):
    s2 = _silu(lax.dot_general(s_ref[...], wsh_ref[...],
                               (((1,), (1,)), ((), ()))) + bsh_ref[...])
    w0c = w0_ref[...][:, :_SDIM]
    t = lax.dot_general(s2, w0c, (((1,), (1,)), ((), ())))

    v2 = v2_ref[...]
    wc = wc_ref[...]
    cl = [jnp.sum(v2[:, c * _VDIM:(c + 1) * _VDIM] * wc, axis=1, keepdims=True)
          for c in range(3)]
    cp = p_ref[...] + jnp.concatenate(cl, axis=1)

    oh = (lax.broadcasted_iota(jnp.int32, (_NG, _N), 0)
          == b_ref[...]).astype(jnp.float32)
    sums = lax.dot_general(oh, cp, (((1,), (0,)), ((), ())))
    cnt = jnp.sum(oh, axis=1, keepdims=True)
    mean = sums / jnp.maximum(cnt, 1.0)
    mg = lax.dot_general(oh, mean, (((0,), (0,)), ((), ())))
    c = cp - mg
    q = jnp.sum(c * c, axis=1, keepdims=True)
    text_ref[...] = jnp.concatenate(
        [t, q, c, jnp.zeros((_N, _TEXT - _SDIM - 4), jnp.float32)], axis=1)

    j2 = j2_ref[...]
    i2 = i2_ref[...]
    k2 = j2 * _N + i2
    rk2 = i2 * _N + j2
    lose = jnp.zeros(k2.shape, jnp.int32)
    for sft in range(1, 16):
        eq = (k2[:, sft:] == k2[:, :16 - sft]).astype(jnp.int32)
        lose = jnp.maximum(
            lose, jnp.concatenate(
                [eq, jnp.zeros((k2.shape[0], sft), jnp.int32)], axis=1))
    k2_ref[...] = k2
    rk2_ref[...] = rk2
    keff2_ref[...] = jnp.where(lose > 0, jnp.int32(0x7FFFFFFF), k2)

    m_ref[...] = lax.dot_general(wb_ref[...], w0c, (((0,), (1,)), ((), ())))
    c0_ref[...] = (b0_ref[...]
                   + lax.dot_general(bb_ref[...], w0c, (((1,), (1,)), ((), ()))))
    wd_ref[...] = jnp.reshape(w0_ref[...][:, _SDIM:], (1, _SDIM))


def _node_call(s, v2, p, b1x, j2, i2, wsh, bsh, wc, wb, bb, w0, b0):
    return pl.pallas_call(
        _node_body,
        out_shape=[
            jax.ShapeDtypeStruct((_N, _TEXT), jnp.float32),
            jax.ShapeDtypeStruct((_E // 16, 16), jnp.int32),
            jax.ShapeDtypeStruct((_E // 16, 16), jnp.int32),
            jax.ShapeDtypeStruct((_E // 16, 16), jnp.int32),
            jax.ShapeDtypeStruct((_EDIM, _SDIM), jnp.float32),
            jax.ShapeDtypeStruct((1, _SDIM), jnp.float32),
            jax.ShapeDtypeStruct((1, _SDIM), jnp.float32),
        ],
    )(s, v2, p, b1x, j2, i2, wsh, bsh, wc, wb, bb, w0, b0)


def _make_scatter():
    mesh = plsc.VectorSubcoreMesh(core_axis_name="c", subcore_axis_name="s")

    @functools.partial(
        pl.kernel, mesh=mesh,
        compiler_params=pltpu.CompilerParams(needs_layout_passes=False),
        out_type=jax.ShapeDtypeStruct((_NKEY,), jnp.int32),
        scratch_types=[
            pltpu.VMEM((_NKEY // _NBUCKET,), jnp.int32),
            pltpu.VMEM((_KW,), jnp.int32),
        ],
    )
    def scatter_k(keff_hbm, table_hbm, tab_v, kbuf):
        wid = lax.axis_index("s") * 2 + lax.axis_index("c")
        bsz = _NKEY // _NBUCKET
        for rnd in range(2):
            bucket = rnd * _NW + wid

            def zero_body(t, carry):
                tab_v[pl.ds(t * 16, 16)] = jnp.zeros((16,), jnp.int32)
                return carry
            lax.fori_loop(0, bsz // 16, zero_body, 0)

            def win_body(w, carry):
                pltpu.sync_copy(keff_hbm.at[pl.ds(w * _KW, _KW)], kbuf)

                def vec_body(t, c2):
                    kv = kbuf[pl.ds(t * 16, 16)]
                    msk = lax.shift_right_logical(kv, 16) == bucket
                    lidx = lax.bitwise_and(kv, 0xFFFF)
                    ids = (w * _KW + t * 16 + 1) + lax.iota(jnp.int32, 16)
                    plsc.store_scatter(tab_v, [lidx], ids, mask=msk)
                    return c2
                lax.fori_loop(0, _KW // 16, vec_body, 0)
                return carry
            lax.fori_loop(0, _E // _KW, win_body, 0)
            pltpu.sync_copy(tab_v, table_hbm.at[pl.ds(bucket * bsz, bsz)])

    return scatter_k


def _make_gather():
    mesh = plsc.VectorSubcoreMesh(core_axis_name="c", subcore_axis_name="s")

    @functools.partial(
        pl.kernel, mesh=mesh,
        compiler_params=pltpu.CompilerParams(use_tc_tiling_on_sc=False),
        out_type=[
            jax.ShapeDtypeStruct((_E, _EDIM), jnp.float32),
            jax.ShapeDtypeStruct((_E, _TEXT), jnp.float32),
        ],
        scratch_types=[
            pltpu.VMEM((_GW,), jnp.int32),
            pltpu.VMEM((_GW,), jnp.int32),
            pltpu.VMEM((_GW,), jnp.int32),
            pltpu.VMEM((_GW,), jnp.int32),
            pltpu.VMEM((_GW,), jnp.int32),
            pltpu.VMEM((_GW,), jnp.int32),
            pltpu.VMEM((_GW,), jnp.int32),
            pltpu.VMEM((_GW,), jnp.int32),
            pltpu.VMEM((_GW, _EDIM), jnp.float32),
            pltpu.VMEM((_GW, _EDIM), jnp.float32),
            pltpu.VMEM((_GW, _TEXT), jnp.float32),
            pltpu.VMEM((_GW, _TEXT), jnp.float32),
            pltpu.SemaphoreType.DMA,
            pltpu.SemaphoreType.DMA,
            pltpu.SemaphoreType.DMA,
            pltpu.SemaphoreType.DMA,
        ],
    )
    def gather_k(table_hbm, keys_hbm, rkeys_hbm, i_hbm, j_hbm, epad_hbm,
                 text_hbm, esym_hbm, g_hbm,
                 kbuf, rkbuf, ibuf, jbuf, fid, rid, idxf, idxr,
                 ef, er, acc_a, acc_b, sem1, sem2, sem3, sem4):
        wid = lax.axis_index("s") * 2 + lax.axis_index("c")

        def win_body(w, carry):
            base = wid * _EPW + w * _GW
            pltpu.sync_copy(keys_hbm.at[pl.ds(base, _GW)], kbuf)
            pltpu.sync_copy(rkeys_hbm.at[pl.ds(base, _GW)], rkbuf)
            pltpu.sync_copy(i_hbm.at[pl.ds(base, _GW)], ibuf)
            pltpu.sync_copy(j_hbm.at[pl.ds(base, _GW)], jbuf)
            cf = pltpu.async_copy(table_hbm.at[kbuf], fid, sem1)
            cr = pltpu.async_copy(table_hbm.at[rkbuf], rid, sem2)
            ca = pltpu.async_copy(text_hbm.at[ibuf], acc_a, sem3)
            cb = pltpu.async_copy(text_hbm.at[jbuf], acc_b, sem4)
            cf.wait()
            cr.wait()

            def idx_body(t, c2):
                f16 = fid[pl.ds(t * 16, 16)] - 1
                idxf[pl.ds(t * 16, 16)] = f16
                r16 = rid[pl.ds(t * 16, 16)]
                zr = _E + lax.bitwise_and(t * 16 + lax.iota(jnp.int32, 16),
                                          127)
                idxr[pl.ds(t * 16, 16)] = jnp.where(r16 > 0, r16 - 1, zr)
                return c2
            lax.fori_loop(0, _GW // 16, idx_body, 0)

            ce = pltpu.async_copy(epad_hbm.at[idxf], ef, sem1)
            cg = pltpu.async_copy(epad_hbm.at[idxr], er, sem2)
            ce.wait()
            cg.wait()
            ca.wait()
            cb.wait()

            def row_body(t, c2):
                er[t] = 0.5 * (ef[t] + er[t])
                for c in range(_TEXT // 16):
                    acc_a[t, pl.ds(c * 16, 16)] = (
                        acc_a[t, pl.ds(c * 16, 16)]
                        + acc_b[t, pl.ds(c * 16, 16)])
                return c2
            lax.fori_loop(0, _GW, row_body, 0)

            pltpu.sync_copy(er, esym_hbm.at[pl.ds(base, _GW)])
            pltpu.sync_copy(acc_a, g_hbm.at[pl.ds(base, _GW)])
            return carry
        lax.fori_loop(0, _EPW // _GW, win_body, 0)

    return gather_k


def _edge_body(g_ref, es_ref, m_ref, c0_ref, wd_ref, w1_ref, b1_ref, o_ref):
    g = g_ref[...]
    ts = g[:, :_SDIM]
    sq = g[:, _SDIM:_SDIM + 1]
    sc = g[:, _SDIM + 1:_SDIM + 4]
    d = 2.0 * sq - jnp.sum(sc * sc, axis=1, keepdims=True)
    pre = (ts
           + lax.dot_general(es_ref[...], m_ref[...], (((1,), (0,)), ((), ())))
           + d * wd_ref[...] + c0_ref[...])
    h = _silu(pre)
    o_ref[...] = lax.dot_general(h, w1_ref[...],
                                 (((1,), (1,)), ((), ()))) + b1_ref[...]


def _edge_call(g, esym, m, c0, wd, w1, b1):
    blk = 1024
    return pl.pallas_call(
        _edge_body,
        grid=(_E // blk,),
        in_specs=[
            pl.BlockSpec((blk, _TEXT), lambda i: (i, 0)),
            pl.BlockSpec((blk, _EDIM), lambda i: (i, 0)),
            pl.BlockSpec((_EDIM, _SDIM), lambda i: (0, 0)),
            pl.BlockSpec((1, _SDIM), lambda i: (0, 0)),
            pl.BlockSpec((1, _SDIM), lambda i: (0, 0)),
            pl.BlockSpec((_NBT, _SDIM), lambda i: (0, 0)),
            pl.BlockSpec((1, _NBT), lambda i: (0, 0)),
        ],
        out_specs=pl.BlockSpec((blk, _NBT), lambda i: (i, 0)),
        out_shape=jax.ShapeDtypeStruct((_E, _NBT), jnp.float32),
    )(g, esym, m, c0, wd, w1, b1)


_scatter_call = _make_scatter()
_gather_call = _make_gather()


def kernel(s, v, p, e, batch, edge_index_global, W_shared, b_shared, W_bond,
           b_bond, W0, b0, W1, b1, Wc):
    ei = edge_index_global.astype(jnp.int32)
    j2 = ei[0].reshape(_E // 16, 16)
    i2 = ei[1].reshape(_E // 16, 16)
    v2 = v.reshape(_N, 3 * _VDIM)
    b1x = batch.astype(jnp.int32).reshape(1, _N)
    e_pad = jnp.concatenate([e, jnp.zeros((128, _EDIM), e.dtype)], axis=0)

    text, k2, keff2, rk2, m, c0, wd = _node_call(
        s, v2, p, b1x, j2, i2, W_shared, b_shared.reshape(1, _SDIM),
        Wc, W_bond, b_bond.reshape(1, _SDIM), W0, b0.reshape(1, _SDIM))

    keys = k2.reshape(_E)
    keff = keff2.reshape(_E)
    rkeys = rk2.reshape(_E)
    iidx = i2.reshape(_E)
    jidx = j2.reshape(_E)

    table = _scatter_call(keff)
    esym, g = _gather_call(table, keys, rkeys, iidx, jidx, e_pad, text)
    return _edge_call(g, esym, m, c0, wd, W1, b1.reshape(1, _NBT))


# speedup vs baseline: 15.0254x; 1.2382x over previous
import functools

import jax
import jax.numpy as jnp
from jax import lax
from jax.experimental import pallas as pl
from jax.experimental.pallas import tpu as pltpu
from jax.experimental.pallas import tpu_sc as plsc

_N = 2048
_E = 65536
_SDIM = 256
_VDIM = 64
_EDIM = 16
_NG = 64
_NBT = 5
_TEXT = 272
_NKEY = _N * _N
_NBUCKET = 64
_NW = 32
_EPW = _E // _NW
_GW = 64
_KW = 2048


def _silu(x):
    return x * (1.0 / (1.0 + jnp.exp(-x)))


def _node_body(s_ref, v2_ref, p_ref, b_ref, j2_ref, i2_ref, wsh_ref, bsh_ref,
               wc_ref, wb_ref, bb_ref, w0_ref, b0_ref,
               text_ref, k2_ref, keff2_ref, rk2_ref, m_ref, c0_ref, wd_ref):
    s2 = _silu(lax.dot_general(s_ref[...], wsh_ref[...],
                               (((1,), (1,)), ((), ()))) + bsh_ref[...])
    w0c = w0_ref[...][:, :_SDIM]
    t = lax.dot_general(s2, w0c, (((1,), (1,)), ((), ())))

    v2 = v2_ref[...]
    wc = wc_ref[...]
    cl = [jnp.sum(v2[:, c * _VDIM:(c + 1) * _VDIM] * wc, axis=1, keepdims=True)
          for c in range(3)]
    cp = p_ref[...] + jnp.concatenate(cl, axis=1)

    oh = (lax.broadcasted_iota(jnp.int32, (_NG, _N), 0)
          == b_ref[...]).astype(jnp.float32)
    sums = lax.dot_general(oh, cp, (((1,), (0,)), ((), ())))
    cnt = jnp.sum(oh, axis=1, keepdims=True)
    mean = sums / jnp.maximum(cnt, 1.0)
    mg = lax.dot_general(oh, mean, (((0,), (0,)), ((), ())))
    c = cp - mg
    q = jnp.sum(c * c, axis=1, keepdims=True)
    text_ref[...] = jnp.concatenate(
        [t, q, c, jnp.zeros((_N, _TEXT - _SDIM - 4), jnp.float32)], axis=1)

    j2 = j2_ref[...]
    i2 = i2_ref[...]
    k2 = j2 * _N + i2
    rk2 = i2 * _N + j2
    lose = jnp.zeros(k2.shape, jnp.int32)
    for sft in range(1, 16):
        eq = (k2[:, sft:] == k2[:, :16 - sft]).astype(jnp.int32)
        lose = jnp.maximum(
            lose, jnp.concatenate(
                [eq, jnp.zeros((k2.shape[0], sft), jnp.int32)], axis=1))
    k2_ref[...] = k2
    rk2_ref[...] = rk2
    keff2_ref[...] = jnp.where(lose > 0, jnp.int32(0x7FFFFFFF), k2)

    m_ref[...] = lax.dot_general(wb_ref[...], w0c, (((0,), (1,)), ((), ())))
    c0_ref[...] = (b0_ref[...]
                   + lax.dot_general(bb_ref[...], w0c, (((1,), (1,)), ((), ()))))
    wd_ref[...] = jnp.reshape(w0_ref[...][:, _SDIM:], (1, _SDIM))


def _node_call(s, v2, p, b1x, j2, i2, wsh, bsh, wc, wb, bb, w0, b0):
    return pl.pallas_call(
        _node_body,
        out_shape=[
            jax.ShapeDtypeStruct((_N, _TEXT), jnp.float32),
            jax.ShapeDtypeStruct((_E // 16, 16), jnp.int32),
            jax.ShapeDtypeStruct((_E // 16, 16), jnp.int32),
            jax.ShapeDtypeStruct((_E // 16, 16), jnp.int32),
            jax.ShapeDtypeStruct((_EDIM, _SDIM), jnp.float32),
            jax.ShapeDtypeStruct((1, _SDIM), jnp.float32),
            jax.ShapeDtypeStruct((1, _SDIM), jnp.float32),
        ],
    )(s, v2, p, b1x, j2, i2, wsh, bsh, wc, wb, bb, w0, b0)


def _make_scatter():
    mesh = plsc.VectorSubcoreMesh(core_axis_name="c", subcore_axis_name="s")

    @functools.partial(
        pl.kernel, mesh=mesh,
        compiler_params=pltpu.CompilerParams(needs_layout_passes=False),
        out_type=jax.ShapeDtypeStruct((_NKEY,), jnp.int32),
        scratch_types=[
            pltpu.VMEM((_NKEY // _NBUCKET,), jnp.int32),
            pltpu.VMEM((_KW,), jnp.int32),
            pltpu.VMEM((_KW,), jnp.int32),
            pltpu.SemaphoreType.DMA,
            pltpu.SemaphoreType.DMA,
        ],
    )
    def scatter_k(keff_hbm, table_hbm, tab_v, kb0, kb1, s0, s1):
        wid = lax.axis_index("s") * 2 + lax.axis_index("c")
        bsz = _NKEY // _NBUCKET
        nwin = _E // _KW
        kbs = [kb0, kb1]
        sms = [s0, s1]
        for rnd in range(2):
            bucket = rnd * _NW + wid

            def zero_body(t, carry):
                for u in range(4):
                    tab_v[pl.ds((t * 4 + u) * 16, 16)] = jnp.zeros(
                        (16,), jnp.int32)
                return carry
            lax.fori_loop(0, bsz // 64, zero_body, 0)

            pltpu.async_copy(keff_hbm.at[pl.ds(0, _KW)], kbs[0], sms[0])

            def pair_body(it, carry):
                for phase in range(2):
                    w = it * 2 + phase
                    kb, sem = kbs[phase], sms[phase]
                    nkb, nsem = kbs[1 - phase], sms[1 - phase]
                    pltpu.make_async_copy(
                        keff_hbm.at[pl.ds(0, _KW)], kb, sem).wait()

                    @pl.when(w + 1 < nwin)
                    def _():
                        pltpu.async_copy(
                            keff_hbm.at[pl.ds((w + 1) * _KW, _KW)], nkb, nsem)

                    def vec_body(t, c2):
                        for u in range(4):
                            tt = t * 4 + u
                            kv = kb[pl.ds(tt * 16, 16)]
                            msk = lax.shift_right_logical(kv, 16) == bucket
                            lidx = lax.bitwise_and(kv, 0xFFFF)
                            ids = ((w * _KW + tt * 16 + 1)
                                   + lax.iota(jnp.int32, 16))
                            plsc.store_scatter(tab_v, [lidx], ids, mask=msk)
                        return c2
                    lax.fori_loop(0, _KW // 64, vec_body, 0)
                return carry
            lax.fori_loop(0, nwin // 2, pair_body, 0)
            pltpu.sync_copy(tab_v, table_hbm.at[pl.ds(bucket * bsz, bsz)])

    return scatter_k


def _make_gather():
    mesh = plsc.VectorSubcoreMesh(core_axis_name="c", subcore_axis_name="s")

    nwin = _EPW // _GW
    set_scratch = [
        pltpu.VMEM((_GW,), jnp.int32),
        pltpu.VMEM((_GW,), jnp.int32),
        pltpu.VMEM((_GW,), jnp.int32),
        pltpu.VMEM((_GW,), jnp.int32),
        pltpu.VMEM((_GW,), jnp.int32),
        pltpu.VMEM((_GW,), jnp.int32),
        pltpu.VMEM((_GW,), jnp.int32),
        pltpu.VMEM((_GW,), jnp.int32),
        pltpu.VMEM((_GW, _EDIM), jnp.float32),
        pltpu.VMEM((_GW, _EDIM), jnp.float32),
        pltpu.VMEM((_GW, _TEXT), jnp.float32),
        pltpu.VMEM((_GW, _TEXT), jnp.float32),
    ] + [pltpu.SemaphoreType.DMA] * 12

    @functools.partial(
        pl.kernel, mesh=mesh,
        compiler_params=pltpu.CompilerParams(use_tc_tiling_on_sc=False),
        out_type=[
            jax.ShapeDtypeStruct((_E, _EDIM), jnp.float32),
            jax.ShapeDtypeStruct((_E, _TEXT), jnp.float32),
        ],
        scratch_types=set_scratch + set_scratch,
    )
    def gather_k(table_hbm, keys_hbm, rkeys_hbm, i_hbm, j_hbm, epad_hbm,
                 text_hbm, esym_hbm, g_hbm, *scr):
        sets = [scr[:24], scr[24:]]
        wid = lax.axis_index("s") * 2 + lax.axis_index("c")
        wbase = wid * _EPW

        def issue_idx(w, p):
            b, sm = sets[p][:12], sets[p][12:]
            base = wbase + w * _GW
            pltpu.async_copy(keys_hbm.at[pl.ds(base, _GW)], b[0], sm[0])
            pltpu.async_copy(rkeys_hbm.at[pl.ds(base, _GW)], b[1], sm[1])
            pltpu.async_copy(i_hbm.at[pl.ds(base, _GW)], b[2], sm[2])
            pltpu.async_copy(j_hbm.at[pl.ds(base, _GW)], b[3], sm[3])

        def wait_idx(p):
            b, sm = sets[p][:12], sets[p][12:]
            for n in range(4):
                pltpu.make_async_copy(
                    keys_hbm.at[pl.ds(0, _GW)], b[n], sm[n]).wait()

        def issue_g1(p):
            b, sm = sets[p][:12], sets[p][12:]
            pltpu.async_copy(table_hbm.at[b[0]], b[4], sm[4])
            pltpu.async_copy(table_hbm.at[b[1]], b[5], sm[5])
            pltpu.async_copy(text_hbm.at[b[2]], b[10], sm[10])
            pltpu.async_copy(text_hbm.at[b[3]], b[11], sm[11])

        def wait_out(p):
            b, sm = sets[p][:12], sets[p][12:]
            pltpu.make_async_copy(
                b[9], esym_hbm.at[pl.ds(0, _GW)], sm[8]).wait()
            pltpu.make_async_copy(
                b[10], g_hbm.at[pl.ds(0, _GW)], sm[9]).wait()

        def c1(w, p):
            b, sm = sets[p][:12], sets[p][12:]
            pltpu.make_async_copy(
                table_hbm.at[pl.ds(0, _GW)], b[4], sm[4]).wait()
            pltpu.make_async_copy(
                table_hbm.at[pl.ds(0, _GW)], b[5], sm[5]).wait()
            base = wbase + w * _GW

            def idx_body(t, c2):
                f16 = b[4][pl.ds(t * 16, 16)] - 1
                b[6][pl.ds(t * 16, 16)] = f16
                r16 = b[5][pl.ds(t * 16, 16)]
                zr = _E + lax.bitwise_and(
                    base + t * 16 + lax.iota(jnp.int32, 16), 127)
                b[7][pl.ds(t * 16, 16)] = jnp.where(r16 > 0, r16 - 1, zr)
                return c2
            lax.fori_loop(0, _GW // 16, idx_body, 0)
            pltpu.async_copy(epad_hbm.at[b[6]], b[8], sm[6])
            pltpu.async_copy(epad_hbm.at[b[7]], b[9], sm[7])

        def c2(w, p):
            b, sm = sets[p][:12], sets[p][12:]
            pltpu.make_async_copy(
                epad_hbm.at[pl.ds(0, _GW)], b[8], sm[6]).wait()
            pltpu.make_async_copy(
                epad_hbm.at[pl.ds(0, _GW)], b[9], sm[7]).wait()
            pltpu.make_async_copy(
                text_hbm.at[pl.ds(0, _GW)], b[10], sm[10]).wait()
            pltpu.make_async_copy(
                text_hbm.at[pl.ds(0, _GW)], b[11], sm[11]).wait()

            def row_body(t, c3):
                b[9][t] = 0.5 * (b[8][t] + b[9][t])
                for c in range(_TEXT // 16):
                    b[10][t, pl.ds(c * 16, 16)] = (
                        b[10][t, pl.ds(c * 16, 16)]
                        + b[11][t, pl.ds(c * 16, 16)])
                return c3
            lax.fori_loop(0, _GW, row_body, 0)
            base = wbase + w * _GW
            pltpu.async_copy(b[9], esym_hbm.at[pl.ds(base, _GW)], sm[8])
            pltpu.async_copy(b[10], g_hbm.at[pl.ds(base, _GW)], sm[9])

        issue_idx(0, 0)
        wait_idx(0)
        issue_g1(0)

        def pair_body(it, carry):
            for phase in range(2):
                w = it * 2 + phase
                p = phase
                q = 1 - p

                @pl.when(w + 1 < nwin)
                def _():
                    issue_idx(w + 1, q)
                c1(w, p)

                @pl.when(w + 1 < nwin)
                def _():
                    @pl.when(w >= 1)
                    def _():
                        wait_out(q)
                    wait_idx(q)
                    issue_g1(q)
                c2(w, p)
            return carry
        lax.fori_loop(0, nwin // 2, pair_body, 0)
        wait_out(0)
        wait_out(1)

    return gather_k


def _edge_body(g_ref, es_ref, m_ref, c0_ref, wd_ref, w1_ref, b1_ref, o_ref):
    g = g_ref[...]
    ts = g[:, :_SDIM]
    sq = g[:, _SDIM:_SDIM + 1]
    sc = g[:, _SDIM + 1:_SDIM + 4]
    d = 2.0 * sq - jnp.sum(sc * sc, axis=1, keepdims=True)
    pre = (ts
           + lax.dot_general(es_ref[...], m_ref[...], (((1,), (0,)), ((), ())))
           + d * wd_ref[...] + c0_ref[...])
    h = _silu(pre)
    o_ref[...] = lax.dot_general(h, w1_ref[...],
                                 (((1,), (1,)), ((), ()))) + b1_ref[...]


def _edge_call(g, esym, m, c0, wd, w1, b1):
    blk = 1024
    return pl.pallas_call(
        _edge_body,
        grid=(_E // blk,),
        in_specs=[
            pl.BlockSpec((blk, _TEXT), lambda i: (i, 0)),
            pl.BlockSpec((blk, _EDIM), lambda i: (i, 0)),
            pl.BlockSpec((_EDIM, _SDIM), lambda i: (0, 0)),
            pl.BlockSpec((1, _SDIM), lambda i: (0, 0)),
            pl.BlockSpec((1, _SDIM), lambda i: (0, 0)),
            pl.BlockSpec((_NBT, _SDIM), lambda i: (0, 0)),
            pl.BlockSpec((1, _NBT), lambda i: (0, 0)),
        ],
        out_specs=pl.BlockSpec((blk, _NBT), lambda i: (i, 0)),
        out_shape=jax.ShapeDtypeStruct((_E, _NBT), jnp.float32),
    )(g, esym, m, c0, wd, w1, b1)


_scatter_call = _make_scatter()
_gather_call = _make_gather()


def kernel(s, v, p, e, batch, edge_index_global, W_shared, b_shared, W_bond,
           b_bond, W0, b0, W1, b1, Wc):
    ei = edge_index_global.astype(jnp.int32)
    j2 = ei[0].reshape(_E // 16, 16)
    i2 = ei[1].reshape(_E // 16, 16)
    v2 = v.reshape(_N, 3 * _VDIM)
    b1x = batch.astype(jnp.int32).reshape(1, _N)
    e_pad = jnp.concatenate([e, jnp.zeros((128, _EDIM), e.dtype)], axis=0)

    text, k2, keff2, rk2, m, c0, wd = _node_call(
        s, v2, p, b1x, j2, i2, W_shared, b_shared.reshape(1, _SDIM),
        Wc, W_bond, b_bond.reshape(1, _SDIM), W0, b0.reshape(1, _SDIM))

    keys = k2.reshape(_E)
    keff = keff2.reshape(_E)
    rkeys = rk2.reshape(_E)
    iidx = i2.reshape(_E)
    jidx = j2.reshape(_E)

    table = _scatter_call(keff)
    esym, g = _gather_call(table, keys, rkeys, iidx, jidx, e_pad, text)
    return _edge_call(g, esym, m, c0, wd, W1, b1.reshape(1, _NBT))
